# baseline (speedup 1.0000x reference)
import functools

import jax
import jax.numpy as jnp
from jax import lax
from jax.experimental import pallas as pl
from jax.experimental.pallas import tpu as pltpu
from jax.experimental.pallas import tpu_sc as plsc

EMBED = 64
BATCH = 16384

NUM_CORES = 2
NUM_SUBCORES = 16
NW = NUM_CORES * NUM_SUBCORES
ROWS_PER_TILE = BATCH // NW
IDX_CHUNK = 128
N_CHUNKS = ROWS_PER_TILE // IDX_CHUNK
LANES = 16


def _sc_gather_mul(x, x_, table):
    mesh = plsc.VectorSubcoreMesh(core_axis_name="c", subcore_axis_name="s")

    @functools.partial(
        pl.kernel,
        out_type=jax.ShapeDtypeStruct((BATCH, EMBED), jnp.float32),
        mesh=mesh,
        compiler_params=pltpu.CompilerParams(use_tc_tiling_on_sc=False),
        scratch_types=[
            pltpu.VMEM((N_CHUNKS, IDX_CHUNK), jnp.int32),
            pltpu.VMEM((N_CHUNKS, IDX_CHUNK), jnp.int32),
            pltpu.VMEM((ROWS_PER_TILE, EMBED), jnp.float32),
            pltpu.VMEM((ROWS_PER_TILE, EMBED), jnp.float32),
            pltpu.SemaphoreType.DMA,
        ],
    )
    def k(x_hbm, x2_hbm, table_hbm, out_hbm, idx1_v, idx2_v, rows1_v,
          rows2_v, sem):
        wid = lax.axis_index("s") * NUM_CORES + lax.axis_index("c")
        base = wid * ROWS_PER_TILE
        for j in range(N_CHUNKS):
            sl = pl.ds(base + j * IDX_CHUNK, IDX_CHUNK)
            pltpu.sync_copy(x_hbm.at[sl], idx1_v.at[j])
            pltpu.sync_copy(x2_hbm.at[sl], idx2_v.at[j])
        copies = []
        for j in range(N_CHUNKS):
            dst = pl.ds(j * IDX_CHUNK, IDX_CHUNK)
            copies.append(pltpu.async_copy(
                table_hbm.at[idx1_v.at[j]], rows1_v.at[dst], sem))
            copies.append(pltpu.async_copy(
                table_hbm.at[idx2_v.at[j]], rows2_v.at[dst], sem))
        for cp in copies:
            cp.wait()

        def body(i, carry):
            for kk in range(EMBED // LANES):
                sl = pl.ds(kk * LANES, LANES)
                rows1_v[i, sl] = rows1_v[i, sl] * rows2_v[i, sl]
            return carry

        lax.fori_loop(0, ROWS_PER_TILE, body, 0)
        pltpu.sync_copy(rows1_v, out_hbm.at[pl.ds(base, ROWS_PER_TILE)])

    return k(x, x_, table)


def _tc_mlp(m, W1, b1r, W2r, b2r):
    BLK = 2048
    H = EMBED // 2

    def mlp_kernel(m_ref, W1_ref, b1_ref, W2_ref, b2_ref, out_ref):
        h = jnp.dot(m_ref[...], W1_ref[...],
                    preferred_element_type=jnp.float32)
        h = jnp.maximum(h + b1_ref[...], 0.0)
        o = jnp.sum(h * W2_ref[...], axis=1, keepdims=True) + b2_ref[...]
        out_ref[...] = 1.0 / (1.0 + jnp.exp(-o))

    return pl.pallas_call(
        mlp_kernel,
        grid=(BATCH // BLK,),
        in_specs=[
            pl.BlockSpec((BLK, EMBED), lambda i: (i, 0)),
            pl.BlockSpec((EMBED, H), lambda i: (0, 0)),
            pl.BlockSpec((1, H), lambda i: (0, 0)),
            pl.BlockSpec((1, H), lambda i: (0, 0)),
            pl.BlockSpec((1, 1), lambda i: (0, 0)),
        ],
        out_specs=pl.BlockSpec((BLK, 1), lambda i: (i, 0)),
        out_shape=jax.ShapeDtypeStruct((BATCH, 1), jnp.float32),
    )(m, W1, b1r, W2r, b2r)


def kernel(x, x_, table, W1, b1, W2, b2):
    x = x.astype(jnp.int32)
    x_ = x_.astype(jnp.int32)
    m = _sc_gather_mul(x, x_, table)
    b1r = b1.reshape(1, EMBED // 2)
    W2r = W2.reshape(1, EMBED // 2)
    b2r = b2.reshape(1, 1)
    return _tc_mlp(m, W1, b1r, W2r, b2r)
